# Initial kernel scaffold; baseline (speedup 1.0000x reference)
import functools

import jax
import jax.numpy as jnp
from jax import lax
from jax.experimental import pallas as pl
from jax.experimental.pallas import tpu as pltpu
from jax.experimental.pallas import tpu_sc as plsc

ANG = 0.52917721067
INV_ANG = 1.0 / ANG
SQRT3 = 1.7320508075688772

N_NODES = 100000
NPAD = 102400
N_EDGES = 3200000
ROW = 80
NROWS = N_EDGES // ROW
NZ = 95
NREF = 5
RC6_ROWS = NZ * NZ
RC6_PAD = 9088
RC6_W = 32

NWORK = 32
ROWS_PER_W = NROWS // NWORK
NSLICE = NPAD // 16
BLK = 2048


def _zero_fill(ref, n):
  @pl.loop(0, n // 16)
  def _(i):
    ref[pl.ds(i * 16, 16)] = jnp.zeros((16,), jnp.float32)


def _sc_mesh():
  return plsc.VectorSubcoreMesh(core_axis_name="c", subcore_axis_name="s")


def _pass1_body(sp_hbm, es_hbm, ed_hbm, d_hbm, cov_hbm, out_hbm,
                sp_sh, cn_sh, cov_v, es_v, ed_v, d_v, sps_v, spd_v, cn_v,
                ob_v):
  cid = lax.axis_index("c")
  sid = lax.axis_index("s")
  wid = sid * 2 + cid

  sl_stage = pl.ds(sid * NSLICE, NSLICE)
  pltpu.sync_copy(sp_hbm.at[sl_stage], sp_sh.at[sl_stage])
  _zero_fill(ob_v, NSLICE)
  pltpu.sync_copy(ob_v, cn_sh.at[sl_stage])
  pltpu.sync_copy(cov_hbm, cov_v)
  plsc.subcore_barrier()

  r0 = wid * ROWS_PER_W

  @pl.loop(0, ROWS_PER_W)
  def _(r):
    row = r0 + r
    pltpu.sync_copy(es_hbm.at[row], es_v)
    pltpu.sync_copy(ed_hbm.at[row], ed_v)
    pltpu.sync_copy(d_hbm.at[row], d_v)
    pltpu.sync_copy(sp_sh.at[es_v], sps_v)
    pltpu.sync_copy(sp_sh.at[ed_v], spd_v)
    for g in range(ROW // 16):
      sl = pl.ds(g * 16, 16)
      rc = (plsc.load_gather(cov_v, [sps_v[sl]]) +
            plsc.load_gather(cov_v, [spd_v[sl]]))
      rij = jnp.maximum(d_v[sl] * INV_ANG, 1e-6)
      x = 16.0 * (rc / rij - 1.0)
      cn_v[sl] = 1.0 / (1.0 + jnp.exp(-x))
    pltpu.sync_copy(cn_v, cn_sh.at[es_v], add=True)

  plsc.subcore_barrier()
  pltpu.sync_copy(cn_sh.at[sl_stage], ob_v)
  pltpu.sync_copy(ob_v, out_hbm.at[cid, sl_stage])


def _run_pass1(sp_p, es2, ed2, d2, cov_p):
  fn = pl.kernel(
      _pass1_body,
      out_type=jax.ShapeDtypeStruct((2, NPAD), jnp.float32),
      mesh=_sc_mesh(),
      scratch_types=[
          pltpu.VMEM_SHARED((NPAD,), jnp.int32),
          pltpu.VMEM_SHARED((NPAD,), jnp.float32),
          pltpu.VMEM((96,), jnp.float32),
          pltpu.VMEM((ROW,), jnp.int32),
          pltpu.VMEM((ROW,), jnp.int32),
          pltpu.VMEM((ROW,), jnp.float32),
          pltpu.VMEM((ROW,), jnp.int32),
          pltpu.VMEM((ROW,), jnp.int32),
          pltpu.VMEM((ROW,), jnp.float32),
          pltpu.VMEM((NSLICE,), jnp.float32),
      ],
  )
  return fn(sp_p, es2, ed2, d2, cov_p)


def _node_body(sp_ref, p0_ref, p1_ref, t_ref, nd_ref):
  sp = sp_ref[...]
  oh = (sp[:, None] == lax.broadcasted_iota(jnp.int32, (BLK, 128), 1)
        ).astype(jnp.float32)
  r = jnp.dot(oh, t_ref[...], preferred_element_type=jnp.float32)
  refcn = r[:, 0:NREF]
  exw = r[:, NREF:2 * NREF]
  g = r[:, 2 * NREF:2 * NREF + 1]
  cn = p0_ref[...] + p1_ref[...]
  mask = refcn >= 0.0
  dcn = refcn - cn[:, None]
  w = jnp.where(mask, jnp.exp(-4.0 * dcn * dcn), 0.0)
  norm = jnp.sum(w, axis=1, keepdims=True)
  wn = jnp.where(mask, w / jnp.maximum(norm, 1e-6), 0.0)
  wf = jnp.where(norm < 1e-6, exw, wn)
  nd_ref[...] = jnp.concatenate(
      [wf, g, jnp.zeros((BLK, 2), jnp.float32)], axis=1)


def _run_node(sp_p, p0, p1, table):
  return pl.pallas_call(
      _node_body,
      grid=(NPAD // BLK,),
      in_specs=[
          pl.BlockSpec((BLK,), lambda i: (i,)),
          pl.BlockSpec((BLK,), lambda i: (i,)),
          pl.BlockSpec((BLK,), lambda i: (i,)),
          pl.BlockSpec((128, 128), lambda i: (0, 0)),
      ],
      out_specs=pl.BlockSpec((BLK, 8), lambda i: (i, 0)),
      out_shape=jax.ShapeDtypeStruct((NPAD, 8), jnp.float32),
  )(sp_p, p0, p1, table)


def _pass2_body(sp_hbm, es_hbm, ed_hbm, d_hbm, sw_hbm, nd_hbm, rc6_hbm,
                out_hbm,
                sp_sh, nd_sh, rc6_sh, e_sh,
                es_v, ed_v, d_v, sw_v, sps_v, spd_v, pair_v,
                nds_v, ndd_v, rc6_v, e_v, ob_v):
  cid = lax.axis_index("c")
  sid = lax.axis_index("s")
  wid = sid * 2 + cid

  sl_stage = pl.ds(sid * NSLICE, NSLICE)
  pltpu.sync_copy(sp_hbm.at[sl_stage], sp_sh.at[sl_stage])
  pltpu.sync_copy(nd_hbm.at[sl_stage], nd_sh.at[sl_stage])
  sl_rc6 = pl.ds(sid * (RC6_PAD // 16), RC6_PAD // 16)
  pltpu.sync_copy(rc6_hbm.at[sl_rc6], rc6_sh.at[sl_rc6])
  _zero_fill(ob_v, NSLICE)
  pltpu.sync_copy(ob_v, e_sh.at[sl_stage])
  plsc.subcore_barrier()

  r0 = wid * ROWS_PER_W

  @pl.loop(0, ROWS_PER_W)
  def _(r):
    row = r0 + r
    pltpu.sync_copy(es_hbm.at[row], es_v)
    pltpu.sync_copy(ed_hbm.at[row], ed_v)
    pltpu.sync_copy(d_hbm.at[row], d_v)
    pltpu.sync_copy(sw_hbm.at[row], sw_v)
    pltpu.sync_copy(sp_sh.at[es_v], sps_v)
    pltpu.sync_copy(sp_sh.at[ed_v], spd_v)
    pltpu.sync_copy(nd_sh.at[es_v], nds_v)
    pltpu.sync_copy(nd_sh.at[ed_v], ndd_v)
    for g in range(ROW // 16):
      sl = pl.ds(g * 16, 16)
      pair_v[sl] = sps_v[sl] * NZ + spd_v[sl]
    pltpu.sync_copy(rc6_sh.at[pair_v], rc6_v)
    for g in range(ROW // 16):
      sl = pl.ds(g * 16, 16)
      r16 = lax.iota(jnp.int32, 16) + (g * 16)
      ws = [plsc.load_gather(nds_v, [r16, jnp.full((16,), a, jnp.int32)])
            for a in range(NREF)]
      wd = [plsc.load_gather(ndd_v, [r16, jnp.full((16,), b, jnp.int32)])
            for b in range(NREF)]
      gs = plsc.load_gather(nds_v, [r16, jnp.full((16,), NREF, jnp.int32)])
      gd = plsc.load_gather(ndd_v, [r16, jnp.full((16,), NREF, jnp.int32)])
      c6 = jnp.zeros((16,), jnp.float32)
      for a in range(NREF):
        for b in range(NREF):
          cab = plsc.load_gather(
              rc6_v, [r16, jnp.full((16,), a * NREF + b, jnp.int32)])
          c6 = c6 + cab * ws[a] * wd[b]
      gg = gs * gd
      qq = 3.0 * gg * gg
      r0d = (0.4 * SQRT3) * gg + 5.0
      rij = jnp.maximum(d_v[sl] * INV_ANG, 1e-6)
      r2 = rij * rij
      r4 = r2 * r2
      r6 = r4 * r2
      r8 = r4 * r4
      p2 = r0d * r0d
      p4 = p2 * p2
      p6 = p4 * p2
      p8 = p4 * p4
      t6 = 1.0 / (r6 + p6)
      t8 = 1.0 / (r8 + p8)
      e_v[sl] = (-0.5) * sw_v[sl] * (c6 * t6 + c6 * qq * t8)
    pltpu.sync_copy(e_v, e_sh.at[es_v], add=True)

  plsc.subcore_barrier()
  pltpu.sync_copy(e_sh.at[sl_stage], ob_v)
  pltpu.sync_copy(ob_v, out_hbm.at[cid, sl_stage])


def _run_pass2(sp_p, es2, ed2, d2, sw2, nd, rc6p):
  fn = pl.kernel(
      _pass2_body,
      out_type=jax.ShapeDtypeStruct((2, NPAD), jnp.float32),
      mesh=_sc_mesh(),
      scratch_types=[
          pltpu.VMEM_SHARED((NPAD,), jnp.int32),
          pltpu.VMEM_SHARED((NPAD, 8), jnp.float32),
          pltpu.VMEM_SHARED((RC6_PAD, RC6_W), jnp.float32),
          pltpu.VMEM_SHARED((NPAD,), jnp.float32),
          pltpu.VMEM((ROW,), jnp.int32),
          pltpu.VMEM((ROW,), jnp.int32),
          pltpu.VMEM((ROW,), jnp.float32),
          pltpu.VMEM((ROW,), jnp.float32),
          pltpu.VMEM((ROW,), jnp.int32),
          pltpu.VMEM((ROW,), jnp.int32),
          pltpu.VMEM((ROW,), jnp.int32),
          pltpu.VMEM((ROW, 8), jnp.float32),
          pltpu.VMEM((ROW, 8), jnp.float32),
          pltpu.VMEM((ROW, RC6_W), jnp.float32),
          pltpu.VMEM((ROW,), jnp.float32),
          pltpu.VMEM((NSLICE,), jnp.float32),
      ],
  )
  return fn(sp_p, es2, ed2, d2, sw2, nd, rc6p)


def _final_body(e0_ref, e1_ref, out_ref):
  out_ref[...] = e0_ref[...] + e1_ref[...]


def _run_final(e0, e1):
  return pl.pallas_call(
      _final_body,
      grid=(NPAD // BLK,),
      in_specs=[
          pl.BlockSpec((BLK,), lambda i: (i,)),
          pl.BlockSpec((BLK,), lambda i: (i,)),
      ],
      out_specs=pl.BlockSpec((BLK,), lambda i: (i,)),
      out_shape=jax.ShapeDtypeStruct((NPAD,), jnp.float32),
  )(e0, e1)


@jax.jit
def kernel(species, edge_src, edge_dst, distances, switch,
           cov_d3, r4r2, ref_cn, ref_c6):
  sp_p = jnp.zeros((NPAD,), jnp.int32).at[:N_NODES].set(species)
  es2 = edge_src.reshape(NROWS, ROW)
  ed2 = edge_dst.reshape(NROWS, ROW)
  d2 = distances.reshape(NROWS, ROW)
  sw2 = switch.reshape(NROWS, ROW)
  cov_p = jnp.zeros((96,), jnp.float32).at[:NZ].set(cov_d3)

  g = jnp.sqrt(r4r2)
  exw = jax.nn.one_hot(jnp.argmax(ref_cn, axis=1), NREF, dtype=jnp.float32)
  table = jnp.zeros((128, 128), jnp.float32)
  table = table.at[:NZ, 0:NREF].set(ref_cn)
  table = table.at[:NZ, NREF:2 * NREF].set(exw)
  table = table.at[:NZ, 2 * NREF].set(g)

  rc6p = jnp.zeros((RC6_PAD, RC6_W), jnp.float32)
  rc6p = rc6p.at[:RC6_ROWS, :NREF * NREF].set(
      ref_c6.reshape(RC6_ROWS, NREF * NREF))

  cnp = _run_pass1(sp_p, es2, ed2, d2, cov_p)
  nd = _run_node(sp_p, cnp[0], cnp[1], table)
  ep = _run_pass2(sp_p, es2, ed2, d2, sw2, nd, rc6p)
  energy = _run_final(ep[0], ep[1])
  return energy[:N_NODES]


# speedup vs baseline: 21.0662x; 21.0662x over previous
import functools

import jax
import jax.numpy as jnp
from jax import lax
from jax.experimental import pallas as pl
from jax.experimental.pallas import tpu as pltpu
from jax.experimental.pallas import tpu_sc as plsc

ANG = 0.52917721067
INV_ANG = 1.0 / ANG
SQRT3 = 1.7320508075688772

N_NODES = 100000
NPAD = 102400
N_EDGES = 3200000
ROW = 80
NROWS = N_EDGES // ROW
NZ = 95
NREF = 5
RC6_ROWS = NZ * NZ
RC6_PAD = 9088
RC6_W = 32

NWORK = 32
ROWS_PER_W = NROWS // NWORK
NSLICE = NPAD // 16
BLK = 2048


def _zero_fill(ref, n):
  @pl.loop(0, n // 16)
  def _(i):
    ref[pl.ds(i * 16, 16)] = jnp.zeros((16,), jnp.float32)


def _sc_mesh():
  return plsc.VectorSubcoreMesh(core_axis_name="c", subcore_axis_name="s")


def _pass1_body(sp_hbm, es_hbm, ed_hbm, d_hbm, cov_hbm, out_hbm,
                cn_sh, cov_v, es_v, ed_v, d_v, sps_v, spd_v, cn_v,
                ob_v):
  cid = lax.axis_index("c")
  sid = lax.axis_index("s")
  wid = sid * 2 + cid

  sl_stage = pl.ds(sid * NSLICE, NSLICE)
  _zero_fill(ob_v, NSLICE)
  pltpu.sync_copy(ob_v, cn_sh.at[sl_stage])
  pltpu.sync_copy(cov_hbm, cov_v)
  plsc.subcore_barrier()

  r0 = wid * ROWS_PER_W

  @pl.loop(0, ROWS_PER_W)
  def _(r):
    row = r0 + r
    pltpu.sync_copy(es_hbm.at[row], es_v)
    pltpu.sync_copy(ed_hbm.at[row], ed_v)
    pltpu.sync_copy(d_hbm.at[row], d_v)
    pltpu.sync_copy(sp_hbm.at[es_v], sps_v)
    pltpu.sync_copy(sp_hbm.at[ed_v], spd_v)
    for g in range(ROW // 16):
      sl = pl.ds(g * 16, 16)
      rc = (plsc.load_gather(cov_v, [sps_v[sl]]) +
            plsc.load_gather(cov_v, [spd_v[sl]]))
      rij = jnp.maximum(d_v[sl] * INV_ANG, 1e-6)
      x = 16.0 * (rc / rij - 1.0)
      cn_v[sl] = 1.0 / (1.0 + jnp.exp(-x))
    pltpu.sync_copy(cn_v, cn_sh.at[es_v], add=True)

  plsc.subcore_barrier()
  pltpu.sync_copy(cn_sh.at[sl_stage], ob_v)
  pltpu.sync_copy(ob_v, out_hbm.at[cid, sl_stage])


def _run_pass1(sp_p, es2, ed2, d2, cov_p):
  fn = pl.kernel(
      _pass1_body,
      out_type=jax.ShapeDtypeStruct((2, NPAD), jnp.float32),
      mesh=_sc_mesh(),
      compiler_params=pltpu.CompilerParams(needs_layout_passes=False, use_tc_tiling_on_sc=False),
      scratch_types=[
          pltpu.VMEM_SHARED((NPAD,), jnp.float32),
          pltpu.VMEM((96,), jnp.float32),
          pltpu.VMEM((ROW,), jnp.int32),
          pltpu.VMEM((ROW,), jnp.int32),
          pltpu.VMEM((ROW,), jnp.float32),
          pltpu.VMEM((ROW,), jnp.int32),
          pltpu.VMEM((ROW,), jnp.int32),
          pltpu.VMEM((ROW,), jnp.float32),
          pltpu.VMEM((NSLICE,), jnp.float32),
      ],
  )
  return fn(sp_p, es2, ed2, d2, cov_p)


def _node_body(sp_ref, p0_ref, p1_ref, t_ref, nd_ref):
  sp = sp_ref[...]
  oh = (sp[:, None] == lax.broadcasted_iota(jnp.int32, (BLK, 128), 1)
        ).astype(jnp.float32)
  r = jnp.dot(oh, t_ref[...], preferred_element_type=jnp.float32)
  refcn = r[:, 0:NREF]
  exw = r[:, NREF:2 * NREF]
  g = r[:, 2 * NREF:2 * NREF + 1]
  cn = p0_ref[...] + p1_ref[...]
  mask = refcn >= 0.0
  dcn = refcn - cn[:, None]
  w = jnp.where(mask, jnp.exp(-4.0 * dcn * dcn), 0.0)
  norm = jnp.sum(w, axis=1, keepdims=True)
  wn = jnp.where(mask, w / jnp.maximum(norm, 1e-6), 0.0)
  wf = jnp.where(norm < 1e-6, exw, wn)
  nd_ref[...] = jnp.concatenate(
      [wf, g, jnp.zeros((BLK, 2), jnp.float32)], axis=1)


def _run_node(sp_p, p0, p1, table):
  return pl.pallas_call(
      _node_body,
      grid=(NPAD // BLK,),
      in_specs=[
          pl.BlockSpec((BLK,), lambda i: (i,)),
          pl.BlockSpec((BLK,), lambda i: (i,)),
          pl.BlockSpec((BLK,), lambda i: (i,)),
          pl.BlockSpec((128, 128), lambda i: (0, 0)),
      ],
      out_specs=pl.BlockSpec((BLK, 8), lambda i: (i, 0)),
      out_shape=jax.ShapeDtypeStruct((NPAD, 8), jnp.float32),
  )(sp_p, p0, p1, table)


def _pass2_body(sp_hbm, es_hbm, ed_hbm, d_hbm, sw_hbm, nd_hbm, rc6_hbm,
                out_hbm,
                e_sh,
                es_v, ed_v, d_v, sw_v, sps_v, spd_v, pair_v,
                nds_v, ndd_v, rc6_v, e_v, ob_v):
  cid = lax.axis_index("c")
  sid = lax.axis_index("s")
  wid = sid * 2 + cid

  sl_stage = pl.ds(sid * NSLICE, NSLICE)
  _zero_fill(ob_v, NSLICE)
  pltpu.sync_copy(ob_v, e_sh.at[sl_stage])
  plsc.subcore_barrier()

  r0 = wid * ROWS_PER_W

  @pl.loop(0, ROWS_PER_W)
  def _(r):
    row = r0 + r
    pltpu.sync_copy(es_hbm.at[row], es_v)
    pltpu.sync_copy(ed_hbm.at[row], ed_v)
    pltpu.sync_copy(d_hbm.at[row], d_v)
    pltpu.sync_copy(sw_hbm.at[row], sw_v)
    pltpu.sync_copy(sp_hbm.at[es_v], sps_v)
    pltpu.sync_copy(sp_hbm.at[ed_v], spd_v)
    pltpu.sync_copy(nd_hbm.at[es_v], nds_v)
    pltpu.sync_copy(nd_hbm.at[ed_v], ndd_v)
    for g in range(ROW // 16):
      sl = pl.ds(g * 16, 16)
      pair_v[sl] = sps_v[sl] * NZ + spd_v[sl]
    pltpu.sync_copy(rc6_hbm.at[pair_v], rc6_v)
    for g in range(ROW // 16):
      sl = pl.ds(g * 16, 16)
      r16 = lax.iota(jnp.int32, 16) + (g * 16)
      ws = [plsc.load_gather(nds_v, [r16, jnp.full((16,), a, jnp.int32)])
            for a in range(NREF)]
      wd = [plsc.load_gather(ndd_v, [r16, jnp.full((16,), b, jnp.int32)])
            for b in range(NREF)]
      gs = plsc.load_gather(nds_v, [r16, jnp.full((16,), NREF, jnp.int32)])
      gd = plsc.load_gather(ndd_v, [r16, jnp.full((16,), NREF, jnp.int32)])
      c6 = jnp.zeros((16,), jnp.float32)
      for a in range(NREF):
        for b in range(NREF):
          cab = plsc.load_gather(
              rc6_v, [r16, jnp.full((16,), a * NREF + b, jnp.int32)])
          c6 = c6 + cab * ws[a] * wd[b]
      gg = gs * gd
      qq = 3.0 * gg * gg
      r0d = (0.4 * SQRT3) * gg + 5.0
      rij = jnp.maximum(d_v[sl] * INV_ANG, 1e-6)
      r2 = rij * rij
      r4 = r2 * r2
      r6 = r4 * r2
      r8 = r4 * r4
      p2 = r0d * r0d
      p4 = p2 * p2
      p6 = p4 * p2
      p8 = p4 * p4
      t6 = 1.0 / (r6 + p6)
      t8 = 1.0 / (r8 + p8)
      e_v[sl] = (-0.5) * sw_v[sl] * (c6 * t6 + c6 * qq * t8)
    pltpu.sync_copy(e_v, e_sh.at[es_v], add=True)

  plsc.subcore_barrier()
  pltpu.sync_copy(e_sh.at[sl_stage], ob_v)
  pltpu.sync_copy(ob_v, out_hbm.at[cid, sl_stage])


def _run_pass2(sp_p, es2, ed2, d2, sw2, nd, rc6p):
  fn = pl.kernel(
      _pass2_body,
      out_type=jax.ShapeDtypeStruct((2, NPAD), jnp.float32),
      mesh=_sc_mesh(),
      compiler_params=pltpu.CompilerParams(needs_layout_passes=False, use_tc_tiling_on_sc=False),
      scratch_types=[
          pltpu.VMEM_SHARED((NPAD,), jnp.float32),
          pltpu.VMEM((ROW,), jnp.int32),
          pltpu.VMEM((ROW,), jnp.int32),
          pltpu.VMEM((ROW,), jnp.float32),
          pltpu.VMEM((ROW,), jnp.float32),
          pltpu.VMEM((ROW,), jnp.int32),
          pltpu.VMEM((ROW,), jnp.int32),
          pltpu.VMEM((ROW,), jnp.int32),
          pltpu.VMEM((ROW, 8), jnp.float32),
          pltpu.VMEM((ROW, 8), jnp.float32),
          pltpu.VMEM((ROW, RC6_W), jnp.float32),
          pltpu.VMEM((ROW,), jnp.float32),
          pltpu.VMEM((NSLICE,), jnp.float32),
      ],
  )
  return fn(sp_p, es2, ed2, d2, sw2, nd, rc6p)


def _final_body(e0_ref, e1_ref, out_ref):
  out_ref[...] = e0_ref[...] + e1_ref[...]


def _run_final(e0, e1):
  return pl.pallas_call(
      _final_body,
      grid=(NPAD // BLK,),
      in_specs=[
          pl.BlockSpec((BLK,), lambda i: (i,)),
          pl.BlockSpec((BLK,), lambda i: (i,)),
      ],
      out_specs=pl.BlockSpec((BLK,), lambda i: (i,)),
      out_shape=jax.ShapeDtypeStruct((NPAD,), jnp.float32),
  )(e0, e1)


@jax.jit
def kernel(species, edge_src, edge_dst, distances, switch,
           cov_d3, r4r2, ref_cn, ref_c6):
  sp_p = jnp.zeros((NPAD,), jnp.int32).at[:N_NODES].set(species)
  es2 = edge_src.reshape(NROWS, ROW)
  ed2 = edge_dst.reshape(NROWS, ROW)
  d2 = distances.reshape(NROWS, ROW)
  sw2 = switch.reshape(NROWS, ROW)
  cov_p = jnp.zeros((96,), jnp.float32).at[:NZ].set(cov_d3)

  g = jnp.sqrt(r4r2)
  exw = jax.nn.one_hot(jnp.argmax(ref_cn, axis=1), NREF, dtype=jnp.float32)
  table = jnp.zeros((128, 128), jnp.float32)
  table = table.at[:NZ, 0:NREF].set(ref_cn)
  table = table.at[:NZ, NREF:2 * NREF].set(exw)
  table = table.at[:NZ, 2 * NREF].set(g)

  rc6p = jnp.zeros((RC6_PAD, RC6_W), jnp.float32)
  rc6p = rc6p.at[:RC6_ROWS, :NREF * NREF].set(
      ref_c6.reshape(RC6_ROWS, NREF * NREF))

  cnp = _run_pass1(sp_p, es2, ed2, d2, cov_p)
  nd = _run_node(sp_p, cnp[0], cnp[1], table)
  ep = _run_pass2(sp_p, es2, ed2, d2, sw2, nd, rc6p)
  energy = _run_final(ep[0], ep[1])
  return energy[:N_NODES]


# speedup vs baseline: 63.7330x; 3.0254x over previous
import jax
import jax.numpy as jnp
from jax import lax
from jax.experimental import pallas as pl
from jax.experimental.pallas import tpu as pltpu
from jax.experimental.pallas import tpu_sc as plsc

ANG = 0.52917721067
INV_ANG = 1.0 / ANG
SQRT3 = 1.7320508075688772

N_NODES = 100000
NPAD = 102400
N_EDGES = 3200000
ROW = 80
NROWS = N_EDGES // ROW
K = 5
NZ = 95
NREF = 5
RC6_ROWS = NZ * NZ
RC6_PAD = 9088
RC6_W = 32

NWORK = 32
ROWS_PER_W = NROWS // NWORK
NSLICE = NPAD // 16
BLK = 2048


def _zero_fill(ref, n):
  @pl.loop(0, n // 16)
  def _(i):
    ref[pl.ds(i * 16, 16)] = jnp.zeros((16,), jnp.float32)


def _sc_mesh():
  return plsc.VectorSubcoreMesh(core_axis_name="c", subcore_axis_name="s")


def _col(c):
  return jnp.full((16,), c, jnp.int32)


def _pass1_body(sp_hbm, ein_hbm, out_hbm,
                cn_sh, cov_v, ein_v, sps_v, spd_v, cn_v, ob_v,
                sem_lin, sem_g):
  cid = lax.axis_index("c")
  sid = lax.axis_index("s")
  wid = sid * 2 + cid

  sl_stage = pl.ds(sid * NSLICE, NSLICE)
  _zero_fill(ob_v, NSLICE)
  pltpu.sync_copy(ob_v, cn_sh.at[sl_stage])
  plsc.subcore_barrier()

  r0 = wid * ROWS_PER_W

  @pl.loop(0, ROWS_PER_W // K)
  def _(it):
    rbase = r0 + it * K
    lin = [pltpu.async_copy(ein_hbm.at[rbase + k], ein_v.at[k], sem_lin)
           for k in range(K)]
    for d in lin:
      d.wait()
    gs = []
    for k in range(K):
      gs.append(pltpu.async_copy(sp_hbm.at[ein_v.at[k, 0]], sps_v.at[k],
                                 sem_g))
      gs.append(pltpu.async_copy(sp_hbm.at[ein_v.at[k, 1]], spd_v.at[k],
                                 sem_g))
    for d in gs:
      d.wait()
    for k in range(K):
      for g in range(ROW // 16):
        sl = pl.ds(g * 16, 16)
        rc = (plsc.load_gather(cov_v, [sps_v[k, sl]]) +
              plsc.load_gather(cov_v, [spd_v[k, sl]]))
        dist = plsc.bitcast(ein_v[k, 2, sl], jnp.float32)
        rij = jnp.maximum(dist * INV_ANG, 1e-6)
        x = 16.0 * (rc / rij - 1.0)
        cn_v[k, sl] = 1.0 / (1.0 + jnp.exp(-x))
    for k in range(K):
      pltpu.sync_copy(cn_v.at[k], cn_sh.at[ein_v.at[k, 0]], add=True)

  plsc.subcore_barrier()
  pltpu.sync_copy(cn_sh.at[sl_stage], ob_v)
  pltpu.sync_copy(ob_v, out_hbm.at[cid, sl_stage])


def _run_pass1(sp_p, ein, cov_p):
  def body(sp_hbm, ein_hbm, cov_hbm, out_hbm, cn_sh, cov_v, ein_v, sps_v,
           spd_v, cn_v, ob_v, sem_lin, sem_g):
    pltpu.sync_copy(cov_hbm, cov_v)
    _pass1_body(sp_hbm, ein_hbm, out_hbm, cn_sh, cov_v, ein_v, sps_v,
                spd_v, cn_v, ob_v, sem_lin, sem_g)

  fn = pl.kernel(
      body,
      out_type=jax.ShapeDtypeStruct((2, NPAD), jnp.float32),
      mesh=_sc_mesh(),
      compiler_params=pltpu.CompilerParams(
          needs_layout_passes=False, use_tc_tiling_on_sc=False),
      scratch_types=[
          pltpu.VMEM_SHARED((NPAD,), jnp.float32),
          pltpu.VMEM((96,), jnp.float32),
          pltpu.VMEM((K, 4, ROW), jnp.int32),
          pltpu.VMEM((K, ROW), jnp.int32),
          pltpu.VMEM((K, ROW), jnp.int32),
          pltpu.VMEM((K, ROW), jnp.float32),
          pltpu.VMEM((NSLICE,), jnp.float32),
          pltpu.SemaphoreType.DMA,
          pltpu.SemaphoreType.DMA,
      ],
  )
  return fn(sp_p, ein, cov_p)


def _node_body(sp_ref, p0_ref, p1_ref, t_ref, nd_ref):
  sp = sp_ref[...]
  oh = (sp[:, None] == lax.broadcasted_iota(jnp.int32, (BLK, 128), 1)
        ).astype(jnp.float32)
  r = jnp.dot(oh, t_ref[...], preferred_element_type=jnp.float32)
  refcn = r[:, 0:NREF]
  exw = r[:, NREF:2 * NREF]
  g = r[:, 2 * NREF:2 * NREF + 1]
  cn = p0_ref[...] + p1_ref[...]
  mask = refcn >= 0.0
  dcn = refcn - cn[:, None]
  w = jnp.where(mask, jnp.exp(-4.0 * dcn * dcn), 0.0)
  norm = jnp.sum(w, axis=1, keepdims=True)
  wn = jnp.where(mask, w / jnp.maximum(norm, 1e-6), 0.0)
  wf = jnp.where(norm < 1e-6, exw, wn)
  spf = lax.bitcast_convert_type(sp, jnp.float32)[:, None]
  nd_ref[...] = jnp.concatenate(
      [wf, g, spf, jnp.zeros((BLK, 1), jnp.float32)], axis=1)


def _run_node(sp_p, p0, p1, table):
  return pl.pallas_call(
      _node_body,
      grid=(NPAD // BLK,),
      in_specs=[
          pl.BlockSpec((BLK,), lambda i: (i,)),
          pl.BlockSpec((BLK,), lambda i: (i,)),
          pl.BlockSpec((BLK,), lambda i: (i,)),
          pl.BlockSpec((128, 128), lambda i: (0, 0)),
      ],
      out_specs=pl.BlockSpec((BLK, 8), lambda i: (i, 0)),
      out_shape=jax.ShapeDtypeStruct((NPAD, 8), jnp.float32),
  )(sp_p, p0, p1, table)


def _pass2_body(ein_hbm, nd_hbm, rc6_hbm, out_hbm,
                e_sh, ein_v, pair_v, nds_v, ndd_v, rc6_v, e_v, ob_v,
                sem_lin, sem_g, sem_r):
  cid = lax.axis_index("c")
  sid = lax.axis_index("s")
  wid = sid * 2 + cid

  sl_stage = pl.ds(sid * NSLICE, NSLICE)
  _zero_fill(ob_v, NSLICE)
  pltpu.sync_copy(ob_v, e_sh.at[sl_stage])
  plsc.subcore_barrier()

  r0 = wid * ROWS_PER_W

  @pl.loop(0, ROWS_PER_W // K)
  def _(it):
    rbase = r0 + it * K
    lin = [pltpu.async_copy(ein_hbm.at[rbase + k], ein_v.at[k], sem_lin)
           for k in range(K)]
    for d in lin:
      d.wait()
    gs = []
    for k in range(K):
      gs.append(pltpu.async_copy(nd_hbm.at[ein_v.at[k, 0]], nds_v.at[k],
                                 sem_g))
      gs.append(pltpu.async_copy(nd_hbm.at[ein_v.at[k, 1]], ndd_v.at[k],
                                 sem_g))
    for d in gs:
      d.wait()
    for k in range(K):
      for g in range(ROW // 16):
        sl = pl.ds(g * 16, 16)
        r16 = lax.iota(jnp.int32, 16) + (g * 16)
        sps = plsc.bitcast(plsc.load_gather(nds_v.at[k], [r16, _col(6)]),
                           jnp.int32)
        spd = plsc.bitcast(plsc.load_gather(ndd_v.at[k], [r16, _col(6)]),
                           jnp.int32)
        pair_v[k, sl] = sps * NZ + spd
    rg = [pltpu.async_copy(rc6_hbm.at[pair_v.at[k]], rc6_v.at[k], sem_r)
          for k in range(K)]
    for d in rg:
      d.wait()
    for k in range(K):
      for g in range(ROW // 16):
        sl = pl.ds(g * 16, 16)
        r16 = lax.iota(jnp.int32, 16) + (g * 16)
        ws = [plsc.load_gather(nds_v.at[k], [r16, _col(a)])
              for a in range(NREF)]
        wd = [plsc.load_gather(ndd_v.at[k], [r16, _col(b)])
              for b in range(NREF)]
        gs16 = plsc.load_gather(nds_v.at[k], [r16, _col(NREF)])
        gd16 = plsc.load_gather(ndd_v.at[k], [r16, _col(NREF)])
        c6 = jnp.zeros((16,), jnp.float32)
        for a in range(NREF):
          for b in range(NREF):
            cab = plsc.load_gather(rc6_v.at[k], [r16, _col(a * NREF + b)])
            c6 = c6 + cab * ws[a] * wd[b]
        gg = gs16 * gd16
        qq = 3.0 * gg * gg
        r0d = (0.4 * SQRT3) * gg + 5.0
        dist = plsc.bitcast(ein_v[k, 2, sl], jnp.float32)
        sw = plsc.bitcast(ein_v[k, 3, sl], jnp.float32)
        rij = jnp.maximum(dist * INV_ANG, 1e-6)
        r2 = rij * rij
        r4 = r2 * r2
        r6 = r4 * r2
        r8 = r4 * r4
        p2 = r0d * r0d
        p4 = p2 * p2
        p6 = p4 * p2
        p8 = p4 * p4
        t6 = 1.0 / (r6 + p6)
        t8 = 1.0 / (r8 + p8)
        e_v[k, sl] = (-0.5) * sw * (c6 * t6 + c6 * qq * t8)
    for k in range(K):
      pltpu.sync_copy(e_v.at[k], e_sh.at[ein_v.at[k, 0]], add=True)

  plsc.subcore_barrier()
  pltpu.sync_copy(e_sh.at[sl_stage], ob_v)
  pltpu.sync_copy(ob_v, out_hbm.at[cid, sl_stage])


def _run_pass2(ein, nd, rc6p):
  fn = pl.kernel(
      _pass2_body,
      out_type=jax.ShapeDtypeStruct((2, NPAD), jnp.float32),
      mesh=_sc_mesh(),
      compiler_params=pltpu.CompilerParams(
          needs_layout_passes=False, use_tc_tiling_on_sc=False),
      scratch_types=[
          pltpu.VMEM_SHARED((NPAD,), jnp.float32),
          pltpu.VMEM((K, 4, ROW), jnp.int32),
          pltpu.VMEM((K, ROW), jnp.int32),
          pltpu.VMEM((K, ROW, 8), jnp.float32),
          pltpu.VMEM((K, ROW, 8), jnp.float32),
          pltpu.VMEM((K, ROW, RC6_W), jnp.float32),
          pltpu.VMEM((K, ROW), jnp.float32),
          pltpu.VMEM((NSLICE,), jnp.float32),
          pltpu.SemaphoreType.DMA,
          pltpu.SemaphoreType.DMA,
          pltpu.SemaphoreType.DMA,
      ],
  )
  return fn(ein, nd, rc6p)


def _final_body(e0_ref, e1_ref, out_ref):
  out_ref[...] = e0_ref[...] + e1_ref[...]


def _run_final(e0, e1):
  return pl.pallas_call(
      _final_body,
      grid=(NPAD // BLK,),
      in_specs=[
          pl.BlockSpec((BLK,), lambda i: (i,)),
          pl.BlockSpec((BLK,), lambda i: (i,)),
      ],
      out_specs=pl.BlockSpec((BLK,), lambda i: (i,)),
      out_shape=jax.ShapeDtypeStruct((NPAD,), jnp.float32),
  )(e0, e1)


@jax.jit
def kernel(species, edge_src, edge_dst, distances, switch,
           cov_d3, r4r2, ref_cn, ref_c6):
  sp_p = jnp.zeros((NPAD,), jnp.int32).at[:N_NODES].set(species)
  ein = jnp.stack([
      edge_src.reshape(NROWS, ROW),
      edge_dst.reshape(NROWS, ROW),
      lax.bitcast_convert_type(distances, jnp.int32).reshape(NROWS, ROW),
      lax.bitcast_convert_type(switch, jnp.int32).reshape(NROWS, ROW),
  ], axis=1)
  cov_p = jnp.zeros((96,), jnp.float32).at[:NZ].set(cov_d3)

  g = jnp.sqrt(r4r2)
  exw = jax.nn.one_hot(jnp.argmax(ref_cn, axis=1), NREF, dtype=jnp.float32)
  table = jnp.zeros((128, 128), jnp.float32)
  table = table.at[:NZ, 0:NREF].set(ref_cn)
  table = table.at[:NZ, NREF:2 * NREF].set(exw)
  table = table.at[:NZ, 2 * NREF].set(g)

  rc6p = jnp.zeros((RC6_PAD, RC6_W), jnp.float32)
  rc6p = rc6p.at[:RC6_ROWS, :NREF * NREF].set(
      ref_c6.reshape(RC6_ROWS, NREF * NREF))

  cnp = _run_pass1(sp_p, ein, cov_p)
  nd = _run_node(sp_p, cnp[0], cnp[1], table)
  ep = _run_pass2(ein, nd, rc6p)
  energy = _run_final(ep[0], ep[1])
  return energy[:N_NODES]


# speedup vs baseline: 78.3593x; 1.2295x over previous
import jax
import jax.numpy as jnp
from jax import lax
from jax.experimental import pallas as pl
from jax.experimental.pallas import tpu as pltpu
from jax.experimental.pallas import tpu_sc as plsc

ANG = 0.52917721067
INV_ANG = 1.0 / ANG
SQRT3 = 1.7320508075688772

N_NODES = 100000
NPAD = 102400
N_EDGES = 3200000
ROW = 80
NROWS = N_EDGES // ROW
K = 5
NZ = 95
NREF = 5
RC6_ROWS = NZ * NZ
RC6_PAD = 9088
RC6_W = 32

NWORK = 32
ROWS_PER_W = NROWS // NWORK
NBATCH = ROWS_PER_W // K
NSLICE = NPAD // 16
BLK = 2048


def _zero_fill(ref, n):
  @pl.loop(0, n // 16)
  def _(i):
    ref[pl.ds(i * 16, 16)] = jnp.zeros((16,), jnp.float32)


def _sc_mesh():
  return plsc.VectorSubcoreMesh(core_axis_name="c", subcore_axis_name="s")


def _col(c):
  return jnp.full((16,), c, jnp.int32)


def _pass1_body(sp_hbm, ein_hbm, cov_hbm, out_hbm,
                cn_sh, cov_v,
                ein0_v, ein1_v, sps0_v, sps1_v, spd0_v, spd1_v,
                cn0_v, cn1_v, ob_v,
                sem_lin, sem_g, sem_s):
  cid = lax.axis_index("c")
  sid = lax.axis_index("s")
  wid = sid * 2 + cid

  sl_stage = pl.ds(sid * NSLICE, NSLICE)
  _zero_fill(ob_v, NSLICE)
  pltpu.sync_copy(ob_v, cn_sh.at[sl_stage])
  pltpu.sync_copy(cov_hbm, cov_v)
  plsc.subcore_barrier()

  r0 = wid * ROWS_PER_W
  ein = (ein0_v, ein1_v)
  sps = (sps0_v, sps1_v)
  spd = (spd0_v, spd1_v)
  cnb = (cn0_v, cn1_v)

  def lin_desc(b, s):
    return pltpu.make_async_copy(ein_hbm.at[pl.ds(r0 + b * K, K)],
                                 ein[s], sem_lin)

  def gather_descs(s):
    ds = []
    for k in range(K):
      ds.append(pltpu.make_async_copy(sp_hbm.at[ein[s].at[k, 0]],
                                      sps[s].at[k], sem_g))
      ds.append(pltpu.make_async_copy(sp_hbm.at[ein[s].at[k, 1]],
                                      spd[s].at[k], sem_g))
    return ds

  def scatter_descs(s):
    return [pltpu.make_async_copy(cnb[s].at[k],
                                  cn_sh.at[ein[s].at[k, 0]], sem_s)
            for k in range(K)]

  def compute(s):
    for k in range(K):
      for g in range(ROW // 16):
        sl = pl.ds(g * 16, 16)
        rc = (plsc.load_gather(cov_v, [sps[s][k, sl]]) +
              plsc.load_gather(cov_v, [spd[s][k, sl]]))
        dist = plsc.bitcast(ein[s][k, 2, sl], jnp.float32)
        rij = jnp.maximum(dist * INV_ANG, 1e-6)
        x = 16.0 * (rc / rij - 1.0)
        cnb[s][k, sl] = 1.0 / (1.0 + jnp.exp(-x))

  def batch(j, b, s, first, last):
    for d in gather_descs(s):
      d.wait()
    if first is not None:
      @pl.when(j > 0)
      def _():
        for d in scatter_descs(1 - s):
          d.wait()
    else:
      for d in scatter_descs(1 - s):
        d.wait()

    def prefetch():
      lin_desc(b + 1, 1 - s).start()
      lin_desc(b + 1, 1 - s).wait()
      for d in gather_descs(1 - s):
        d.start()
    if last is not None:
      @pl.when(j < (NBATCH // 2 - 1))
      def _():
        prefetch()
    else:
      prefetch()
    compute(s)
    for d in scatter_descs(s):
      d.start(add=True)

  lin_desc(0, 0).start()
  lin_desc(0, 0).wait()
  for d in gather_descs(0):
    d.start()

  @pl.loop(0, NBATCH // 2)
  def _(j):
    b0 = j * 2
    batch(j, b0, 0, first=True, last=None)
    batch(j, b0 + 1, 1, first=None, last=True)

  for d in scatter_descs(1):
    d.wait()

  plsc.subcore_barrier()
  pltpu.sync_copy(cn_sh.at[sl_stage], ob_v)
  pltpu.sync_copy(ob_v, out_hbm.at[cid, sl_stage])


def _run_pass1(sp_p, ein, cov_p):
  fn = pl.kernel(
      _pass1_body,
      out_type=jax.ShapeDtypeStruct((2, NPAD), jnp.float32),
      mesh=_sc_mesh(),
      compiler_params=pltpu.CompilerParams(
          needs_layout_passes=False, use_tc_tiling_on_sc=False),
      scratch_types=[
          pltpu.VMEM_SHARED((NPAD,), jnp.float32),
          pltpu.VMEM((96,), jnp.float32),
          pltpu.VMEM((K, 4, ROW), jnp.int32),
          pltpu.VMEM((K, 4, ROW), jnp.int32),
          pltpu.VMEM((K, ROW), jnp.int32),
          pltpu.VMEM((K, ROW), jnp.int32),
          pltpu.VMEM((K, ROW), jnp.int32),
          pltpu.VMEM((K, ROW), jnp.int32),
          pltpu.VMEM((K, ROW), jnp.float32),
          pltpu.VMEM((K, ROW), jnp.float32),
          pltpu.VMEM((NSLICE,), jnp.float32),
          pltpu.SemaphoreType.DMA,
          pltpu.SemaphoreType.DMA,
          pltpu.SemaphoreType.DMA,
      ],
  )
  return fn(sp_p, ein, cov_p)


def _node_body(sp_ref, p0_ref, p1_ref, t_ref, nd_ref):
  sp = sp_ref[...]
  oh = (sp[:, None] == lax.broadcasted_iota(jnp.int32, (BLK, 128), 1)
        ).astype(jnp.float32)
  r = jnp.dot(oh, t_ref[...], preferred_element_type=jnp.float32)
  refcn = r[:, 0:NREF]
  exw = r[:, NREF:2 * NREF]
  g = r[:, 2 * NREF:2 * NREF + 1]
  cn = p0_ref[...] + p1_ref[...]
  mask = refcn >= 0.0
  dcn = refcn - cn[:, None]
  w = jnp.where(mask, jnp.exp(-4.0 * dcn * dcn), 0.0)
  norm = jnp.sum(w, axis=1, keepdims=True)
  wn = jnp.where(mask, w / jnp.maximum(norm, 1e-6), 0.0)
  wf = jnp.where(norm < 1e-6, exw, wn)
  spf = lax.bitcast_convert_type(sp, jnp.float32)[:, None]
  nd_ref[...] = jnp.concatenate(
      [wf, g, spf, jnp.zeros((BLK, 1), jnp.float32)], axis=1)


def _run_node(sp_p, p0, p1, table):
  return pl.pallas_call(
      _node_body,
      grid=(NPAD // BLK,),
      in_specs=[
          pl.BlockSpec((BLK,), lambda i: (i,)),
          pl.BlockSpec((BLK,), lambda i: (i,)),
          pl.BlockSpec((BLK,), lambda i: (i,)),
          pl.BlockSpec((128, 128), lambda i: (0, 0)),
      ],
      out_specs=pl.BlockSpec((BLK, 8), lambda i: (i, 0)),
      out_shape=jax.ShapeDtypeStruct((NPAD, 8), jnp.float32),
  )(sp_p, p0, p1, table)


def _pass2_body(ein_hbm, nd_hbm, rc6_hbm, out_hbm,
                e_sh,
                ein0_v, ein1_v, nds0_v, nds1_v, ndd0_v, ndd1_v,
                pair_v, rc6_v, e0_v, e1_v, ob_v,
                sem_lin, sem_g, sem_r, sem_s):
  cid = lax.axis_index("c")
  sid = lax.axis_index("s")
  wid = sid * 2 + cid

  sl_stage = pl.ds(sid * NSLICE, NSLICE)
  _zero_fill(ob_v, NSLICE)
  pltpu.sync_copy(ob_v, e_sh.at[sl_stage])
  plsc.subcore_barrier()

  r0 = wid * ROWS_PER_W
  ein = (ein0_v, ein1_v)
  nds = (nds0_v, nds1_v)
  ndd = (ndd0_v, ndd1_v)
  ev = (e0_v, e1_v)

  def lin_desc(b, s):
    return pltpu.make_async_copy(ein_hbm.at[pl.ds(r0 + b * K, K)],
                                 ein[s], sem_lin)

  def gather_descs(s):
    ds = []
    for k in range(K):
      ds.append(pltpu.make_async_copy(nd_hbm.at[ein[s].at[k, 0]],
                                      nds[s].at[k], sem_g))
      ds.append(pltpu.make_async_copy(nd_hbm.at[ein[s].at[k, 1]],
                                      ndd[s].at[k], sem_g))
    return ds

  def rc6_descs(s):
    return [pltpu.make_async_copy(rc6_hbm.at[pair_v.at[k]],
                                  rc6_v.at[k], sem_r)
            for k in range(K)]

  def scatter_descs(s):
    return [pltpu.make_async_copy(ev[s].at[k],
                                  e_sh.at[ein[s].at[k, 0]], sem_s)
            for k in range(K)]

  def pair_compute(s):
    for k in range(K):
      for g in range(ROW // 16):
        sl = pl.ds(g * 16, 16)
        r16 = lax.iota(jnp.int32, 16) + (g * 16)
        sps = plsc.bitcast(
            plsc.load_gather(nds[s].at[k], [r16, _col(6)]), jnp.int32)
        spd = plsc.bitcast(
            plsc.load_gather(ndd[s].at[k], [r16, _col(6)]), jnp.int32)
        pair_v[k, sl] = sps * NZ + spd

  def compute(s):
    for k in range(K):
      for g in range(ROW // 16):
        sl = pl.ds(g * 16, 16)
        r16 = lax.iota(jnp.int32, 16) + (g * 16)
        ws = [plsc.load_gather(nds[s].at[k], [r16, _col(a)])
              for a in range(NREF)]
        wd = [plsc.load_gather(ndd[s].at[k], [r16, _col(b)])
              for b in range(NREF)]
        gs16 = plsc.load_gather(nds[s].at[k], [r16, _col(NREF)])
        gd16 = plsc.load_gather(ndd[s].at[k], [r16, _col(NREF)])
        c6 = jnp.zeros((16,), jnp.float32)
        for a in range(NREF):
          for b in range(NREF):
            cab = plsc.load_gather(rc6_v.at[k], [r16, _col(a * NREF + b)])
            c6 = c6 + cab * ws[a] * wd[b]
        gg = gs16 * gd16
        qq = 3.0 * gg * gg
        r0d = (0.4 * SQRT3) * gg + 5.0
        dist = plsc.bitcast(ein[s][k, 2, sl], jnp.float32)
        sw = plsc.bitcast(ein[s][k, 3, sl], jnp.float32)
        rij = jnp.maximum(dist * INV_ANG, 1e-6)
        r2 = rij * rij
        r4 = r2 * r2
        r6 = r4 * r2
        r8 = r4 * r4
        p2 = r0d * r0d
        p4 = p2 * p2
        p6 = p4 * p2
        p8 = p4 * p4
        t6 = 1.0 / (r6 + p6)
        t8 = 1.0 / (r8 + p8)
        ev[s][k, sl] = (-0.5) * sw * (c6 * t6 + c6 * qq * t8)

  def batch(j, b, s, first, last):
    for d in gather_descs(s):
      d.wait()
    pair_compute(s)
    for d in rc6_descs(s):
      d.start()
    if first is not None:
      @pl.when(j > 0)
      def _():
        for d in scatter_descs(1 - s):
          d.wait()
    else:
      for d in scatter_descs(1 - s):
        d.wait()

    def prefetch():
      lin_desc(b + 1, 1 - s).start()
      lin_desc(b + 1, 1 - s).wait()
      for d in gather_descs(1 - s):
        d.start()
    if last is not None:
      @pl.when(j < (NBATCH // 2 - 1))
      def _():
        prefetch()
    else:
      prefetch()
    for d in rc6_descs(s):
      d.wait()
    compute(s)
    for d in scatter_descs(s):
      d.start(add=True)

  lin_desc(0, 0).start()
  lin_desc(0, 0).wait()
  for d in gather_descs(0):
    d.start()

  @pl.loop(0, NBATCH // 2)
  def _(j):
    b0 = j * 2
    batch(j, b0, 0, first=True, last=None)
    batch(j, b0 + 1, 1, first=None, last=True)

  for d in scatter_descs(1):
    d.wait()

  plsc.subcore_barrier()
  pltpu.sync_copy(e_sh.at[sl_stage], ob_v)
  pltpu.sync_copy(ob_v, out_hbm.at[cid, sl_stage])


def _run_pass2(ein, nd, rc6p):
  fn = pl.kernel(
      _pass2_body,
      out_type=jax.ShapeDtypeStruct((2, NPAD), jnp.float32),
      mesh=_sc_mesh(),
      compiler_params=pltpu.CompilerParams(
          needs_layout_passes=False, use_tc_tiling_on_sc=False),
      scratch_types=[
          pltpu.VMEM_SHARED((NPAD,), jnp.float32),
          pltpu.VMEM((K, 4, ROW), jnp.int32),
          pltpu.VMEM((K, 4, ROW), jnp.int32),
          pltpu.VMEM((K, ROW, 8), jnp.float32),
          pltpu.VMEM((K, ROW, 8), jnp.float32),
          pltpu.VMEM((K, ROW, 8), jnp.float32),
          pltpu.VMEM((K, ROW, 8), jnp.float32),
          pltpu.VMEM((K, ROW), jnp.int32),
          pltpu.VMEM((K, ROW, RC6_W), jnp.float32),
          pltpu.VMEM((K, ROW), jnp.float32),
          pltpu.VMEM((K, ROW), jnp.float32),
          pltpu.VMEM((NSLICE,), jnp.float32),
          pltpu.SemaphoreType.DMA,
          pltpu.SemaphoreType.DMA,
          pltpu.SemaphoreType.DMA,
          pltpu.SemaphoreType.DMA,
      ],
  )
  return fn(ein, nd, rc6p)


def _final_body(e0_ref, e1_ref, out_ref):
  out_ref[...] = e0_ref[...] + e1_ref[...]


def _run_final(e0, e1):
  return pl.pallas_call(
      _final_body,
      grid=(NPAD // BLK,),
      in_specs=[
          pl.BlockSpec((BLK,), lambda i: (i,)),
          pl.BlockSpec((BLK,), lambda i: (i,)),
      ],
      out_specs=pl.BlockSpec((BLK,), lambda i: (i,)),
      out_shape=jax.ShapeDtypeStruct((NPAD,), jnp.float32),
  )(e0, e1)


@jax.jit
def kernel(species, edge_src, edge_dst, distances, switch,
           cov_d3, r4r2, ref_cn, ref_c6):
  sp_p = jnp.zeros((NPAD,), jnp.int32).at[:N_NODES].set(species)
  ein = jnp.stack([
      edge_src.reshape(NROWS, ROW),
      edge_dst.reshape(NROWS, ROW),
      lax.bitcast_convert_type(distances, jnp.int32).reshape(NROWS, ROW),
      lax.bitcast_convert_type(switch, jnp.int32).reshape(NROWS, ROW),
  ], axis=1)
  cov_p = jnp.zeros((96,), jnp.float32).at[:NZ].set(cov_d3)

  g = jnp.sqrt(r4r2)
  exw = jax.nn.one_hot(jnp.argmax(ref_cn, axis=1), NREF, dtype=jnp.float32)
  table = jnp.zeros((128, 128), jnp.float32)
  table = table.at[:NZ, 0:NREF].set(ref_cn)
  table = table.at[:NZ, NREF:2 * NREF].set(exw)
  table = table.at[:NZ, 2 * NREF].set(g)

  rc6p = jnp.zeros((RC6_PAD, RC6_W), jnp.float32)
  rc6p = rc6p.at[:RC6_ROWS, :NREF * NREF].set(
      ref_c6.reshape(RC6_ROWS, NREF * NREF))

  cnp = _run_pass1(sp_p, ein, cov_p)
  nd = _run_node(sp_p, cnp[0], cnp[1], table)
  ep = _run_pass2(ein, nd, rc6p)
  energy = _run_final(ep[0], ep[1])
  return energy[:N_NODES]
